# baseline (speedup 1.0000x reference)
import jax
import jax.numpy as jnp
from jax.experimental import pallas as pl
from jax.experimental.pallas import tpu as pltpu


def _gru_body(C, BG, H, O, NC):
    bf = jnp.bfloat16

    def body(x_ref, h0_ref,
             w0i_ref, wcat_ref, w1h_ref,
             b0rz_ref, b0in_ref, bh0n_ref,
             b1rz_ref, b1in_ref, bh1n_ref,
             fcw_ref, fcb_ref,
             y_ref, hT_ref):
        c = pl.program_id(1)
        first = c == 0

        @pl.when(first)
        def _():
            hT_ref[...] = h0_ref[...]

        b0rz = jnp.broadcast_to(b0rz_ref[...], (BG, 2 * H))
        b0in = jnp.broadcast_to(b0in_ref[...], (BG, H))
        bh0n = jnp.broadcast_to(bh0n_ref[...], (BG, H))
        b1rz = jnp.broadcast_to(b1rz_ref[...], (BG, 2 * H))
        b1in = jnp.broadcast_to(b1in_ref[...], (BG, H))
        bh1n = jnp.broadcast_to(bh1n_ref[...], (BG, H))

        def gru_step(h, gi, gh, brz, bin_, bhn):
            a = gi[:, 0:2 * H] + gh[:, 0:2 * H] + brz
            r = 0.5 * jnp.tanh(a[:, 0:H]) + 0.5
            z = 0.5 * jnp.tanh(a[:, H:2 * H]) + 0.5
            n = jnp.tanh(gi[:, 2 * H:] + bin_ + r * (gh[:, 2 * H:] + bhn))
            return n + z * (h - n)

        h0 = hT_ref[0]
        h1 = hT_ref[1]

        for t in range(C):
            xt = x_ref[t].astype(bf)
            h0b = h0.astype(bf)
            h1b = h1.astype(bf)
            gi0 = jnp.dot(xt, w0i_ref[...],
                          preferred_element_type=jnp.float32)
            cat = jnp.dot(h0b, wcat_ref[...],
                          preferred_element_type=jnp.float32)
            gh1 = jnp.dot(h1b, w1h_ref[...],
                          preferred_element_type=jnp.float32)

            h0_new = gru_step(h0, gi0, cat[:, 0:3 * H], b0rz, b0in, bh0n)
            h1_new = gru_step(h1, cat[:, 3 * H:], gh1, b1rz, b1in, bh1n)
            if t == 0:
                h1_new = jnp.where(first, h1, h1_new)
            h0 = h0_new
            h1 = h1_new

        hT_ref[0] = h0
        hT_ref[1] = h1

        @pl.when(c == NC - 1)
        def _():
            h0b = h0.astype(bf)
            h1b = h1.astype(bf)
            gi1 = jnp.dot(h0b, wcat_ref[...][:, 3 * H:],
                          preferred_element_type=jnp.float32)
            gh1 = jnp.dot(h1b, w1h_ref[...],
                          preferred_element_type=jnp.float32)
            h1f = gru_step(h1, gi1, gh1, b1rz, b1in, bh1n)
            hT_ref[1] = h1f
            y_ref[...] = (
                jnp.dot(h1f.astype(bf), fcw_ref[...],
                        preferred_element_type=jnp.float32)
                + fcb_ref[...])

    return body


@jax.jit
def _seq_gru(x, h0, l0_w_ih, l0_w_hh, l0_b_ih, l0_b_hh,
             l1_w_ih, l1_w_hh, l1_b_ih, l1_b_hh, fc_w_p, fc_b_p):
    T, B, I = x.shape
    L, _, H = h0.shape
    O = fc_w_p.shape[1]

    BG = min(B, 256)
    NB = B // BG
    C = 64
    while T % C:
        C //= 2
    NC = T // C

    bf = jnp.bfloat16
    NGH = 3 * H

    def half_rz(w):
        return jnp.concatenate([0.5 * w[:, :2 * H], w[:, 2 * H:NGH]], axis=1)

    w0i = half_rz(l0_w_ih).astype(bf)
    w0h = half_rz(l0_w_hh).astype(bf)
    w1i = half_rz(l1_w_ih).astype(bf)
    w1h = half_rz(l1_w_hh).astype(bf)
    wcat = jnp.concatenate([w0h, w1i], axis=1)

    b0rz = 0.5 * (l0_b_ih[:, :2 * H] + l0_b_hh[:, :2 * H])
    b0in = l0_b_ih[:, 2 * H:NGH]
    bh0n = l0_b_hh[:, 2 * H:NGH]
    b1rz = 0.5 * (l1_b_ih[:, :2 * H] + l1_b_hh[:, :2 * H])
    b1in = l1_b_ih[:, 2 * H:NGH]
    bh1n = l1_b_hh[:, 2 * H:NGH]

    params = [w0i, wcat, w1h, b0rz, b0in, bh0n, b1rz, b1in, bh1n,
              fc_w_p.astype(bf), fc_b_p]

    in_specs = [
        pl.BlockSpec((C, BG, I), lambda b, c: (c, b, 0)),
        pl.BlockSpec((L, BG, H), lambda b, c: (0, b, 0)),
    ]
    for w in params:
        in_specs.append(pl.BlockSpec(w.shape, lambda b, c: (0, 0)))

    out_shape = (jax.ShapeDtypeStruct((B, O), jnp.float32),
                 jax.ShapeDtypeStruct((L, B, H), jnp.float32))
    out_specs = (pl.BlockSpec((BG, O), lambda b, c: (b, 0)),
                 pl.BlockSpec((L, BG, H), lambda b, c: (0, b, 0)))

    logits, hT = pl.pallas_call(
        _gru_body(C, BG, H, O, NC),
        grid=(NB, NC),
        in_specs=in_specs,
        out_specs=out_specs,
        out_shape=out_shape,
        compiler_params=pltpu.CompilerParams(
            dimension_semantics=("parallel", "arbitrary"),
            vmem_limit_bytes=64 << 20),
    )(x, h0, *params)

    y = jax.nn.log_softmax(logits, axis=0)
    return y, hT


def kernel(x, h0, l0_w_ih, l0_w_hh, l0_b_ih, l0_b_hh,
           l1_w_ih, l1_w_hh, l1_b_ih, l1_b_hh, fc_w_p, fc_b_p):
    return _seq_gru(x, h0, l0_w_ih, l0_w_hh, l0_b_ih, l0_b_hh,
                    l1_w_ih, l1_w_hh, l1_b_ih, l1_b_hh, fc_w_p, fc_b_p)


# speedup vs baseline: 1.0530x; 1.0530x over previous
import jax
import jax.numpy as jnp
from jax.experimental import pallas as pl
from jax.experimental.pallas import tpu as pltpu


def _gru_body(C, BG, H, O, NC):
    bf = jnp.bfloat16

    def body(x_ref, h0_ref,
             w0i_ref, wcat_ref, w1h_ref,
             b0rz_ref, b0in_ref, bh0n_ref,
             b1rz_ref, b1in_ref, bh1n_ref,
             fcw_ref, fcb_ref,
             y_ref, hT_ref):
        c = pl.program_id(1)
        first = c == 0

        @pl.when(first)
        def _():
            hT_ref[...] = h0_ref[...]

        b0rz = jnp.broadcast_to(b0rz_ref[...], (BG, 2 * H))
        b0in = jnp.broadcast_to(b0in_ref[...], (BG, H))
        bh0n = jnp.broadcast_to(bh0n_ref[...], (BG, H))
        b1rz = jnp.broadcast_to(b1rz_ref[...], (BG, 2 * H))
        b1in = jnp.broadcast_to(b1in_ref[...], (BG, H))
        bh1n = jnp.broadcast_to(bh1n_ref[...], (BG, H))

        def gru_step(h, gi, gh, brz, bin_, bhn):
            a = gi[:, 0:2 * H] + gh[:, 0:2 * H] + brz
            r = 0.5 * jnp.tanh(a[:, 0:H]) + 0.5
            z = 0.5 * jnp.tanh(a[:, H:2 * H]) + 0.5
            n = jnp.tanh(gi[:, 2 * H:] + bin_ + r * (gh[:, 2 * H:] + bhn))
            return n + z * (h - n)

        h0 = hT_ref[0]
        h1 = hT_ref[1]

        for t in range(C):
            xt = x_ref[t].astype(bf)
            h0b = h0.astype(bf)
            h1b = h1.astype(bf)
            gi0 = jnp.dot(xt, w0i_ref[...],
                          preferred_element_type=jnp.float32)
            cat = jnp.dot(h0b, wcat_ref[...],
                          preferred_element_type=jnp.float32)
            gh1 = jnp.dot(h1b, w1h_ref[...],
                          preferred_element_type=jnp.float32)

            h0_new = gru_step(h0, gi0, cat[:, 0:3 * H], b0rz, b0in, bh0n)
            h1_new = gru_step(h1, cat[:, 3 * H:], gh1, b1rz, b1in, bh1n)
            if t == 0:
                h1_new = jnp.where(first, h1, h1_new)
            h0 = h0_new
            h1 = h1_new

        hT_ref[0] = h0
        hT_ref[1] = h1

        @pl.when(c == NC - 1)
        def _():
            h0b = h0.astype(bf)
            h1b = h1.astype(bf)
            gi1 = jnp.dot(h0b, wcat_ref[...][:, 3 * H:],
                          preferred_element_type=jnp.float32)
            gh1 = jnp.dot(h1b, w1h_ref[...],
                          preferred_element_type=jnp.float32)
            h1f = gru_step(h1, gi1, gh1, b1rz, b1in, bh1n)
            hT_ref[1] = h1f
            y_ref[...] = (
                jnp.dot(h1f.astype(bf), fcw_ref[...],
                        preferred_element_type=jnp.float32)
                + fcb_ref[...])

    return body


@jax.jit
def _seq_gru(x, h0, l0_w_ih, l0_w_hh, l0_b_ih, l0_b_hh,
             l1_w_ih, l1_w_hh, l1_b_ih, l1_b_hh, fc_w_p, fc_b_p):
    T, B, I = x.shape
    L, _, H = h0.shape
    O = fc_w_p.shape[1]

    BG = 128 if B >= 128 else B
    NB = B // BG
    C = 128
    while T % C:
        C //= 2
    NC = T // C

    bf = jnp.bfloat16
    NGH = 3 * H

    def half_rz(w):
        return jnp.concatenate([0.5 * w[:, :2 * H], w[:, 2 * H:NGH]], axis=1)

    w0i = half_rz(l0_w_ih).astype(bf)
    w0h = half_rz(l0_w_hh).astype(bf)
    w1i = half_rz(l1_w_ih).astype(bf)
    w1h = half_rz(l1_w_hh).astype(bf)
    wcat = jnp.concatenate([w0h, w1i], axis=1)

    b0rz = 0.5 * (l0_b_ih[:, :2 * H] + l0_b_hh[:, :2 * H])
    b0in = l0_b_ih[:, 2 * H:NGH]
    bh0n = l0_b_hh[:, 2 * H:NGH]
    b1rz = 0.5 * (l1_b_ih[:, :2 * H] + l1_b_hh[:, :2 * H])
    b1in = l1_b_ih[:, 2 * H:NGH]
    bh1n = l1_b_hh[:, 2 * H:NGH]

    params = [w0i, wcat, w1h, b0rz, b0in, bh0n, b1rz, b1in, bh1n,
              fc_w_p.astype(bf), fc_b_p]

    in_specs = [
        pl.BlockSpec((C, BG, I), lambda b, c: (c, b, 0)),
        pl.BlockSpec((L, BG, H), lambda b, c: (0, b, 0)),
    ]
    for w in params:
        in_specs.append(pl.BlockSpec(w.shape, lambda b, c: (0, 0)))

    out_shape = (jax.ShapeDtypeStruct((B, O), jnp.float32),
                 jax.ShapeDtypeStruct((L, B, H), jnp.float32))
    out_specs = (pl.BlockSpec((BG, O), lambda b, c: (b, 0)),
                 pl.BlockSpec((L, BG, H), lambda b, c: (0, b, 0)))

    logits, hT = pl.pallas_call(
        _gru_body(C, BG, H, O, NC),
        grid=(NB, NC),
        in_specs=in_specs,
        out_specs=out_specs,
        out_shape=out_shape,
        compiler_params=pltpu.CompilerParams(
            dimension_semantics=("parallel", "arbitrary"),
            vmem_limit_bytes=64 << 20),
    )(x, h0, *params)

    y = jax.nn.log_softmax(logits, axis=0)
    return y, hT


def kernel(x, h0, l0_w_ih, l0_w_hh, l0_b_ih, l0_b_hh,
           l1_w_ih, l1_w_hh, l1_b_ih, l1_b_hh, fc_w_p, fc_b_p):
    return _seq_gru(x, h0, l0_w_ih, l0_w_hh, l0_b_ih, l0_b_hh,
                    l1_w_ih, l1_w_hh, l1_b_ih, l1_b_hh, fc_w_p, fc_b_p)
